# baseline (speedup 1.0000x reference)
import jax
import jax.numpy as jnp
from jax.experimental import pallas as pl
from jax.experimental.pallas import tpu as pltpu

_TT = 128


def _body(enc_ref, dec_ref, we_ref, be_ref, wd_ref, bd_ref, wf_ref, bf_ref,
          out_ref, c0_ref, wft_ref, x_ref):
    tt, e = enc_ref.shape[1], enc_ref.shape[2]
    u = dec_ref.shape[1]
    in_f = we_ref.shape[0]
    v = wf_ref.shape[0]

    first = (pl.program_id(0) == 0) & (pl.program_id(1) == 0)

    @pl.when(first)
    def _init():
        bound = jnp.sum(jnp.abs(wf_ref[...]), axis=1) + jnp.abs(bf_ref[0])
        c0_ref[0, 0] = jnp.max(bound)
        wft_ref[...] = jnp.transpose(wf_ref[...], (1, 0)).astype(jnp.bfloat16)

    c0 = c0_ref[0, 0]

    enc_x = enc_ref[0]
    dec_x = dec_ref[0]
    enc = jax.lax.dot_general(
        enc_x, we_ref[...], (((1,), (1,)), ((), ())),
        preferred_element_type=jnp.float32) + be_ref[0]
    dec = jax.lax.dot_general(
        dec_x, wd_ref[...], (((1,), (1,)), ((), ())),
        preferred_element_type=jnp.float32) + bd_ref[0]

    comb = enc[:, None, :] + dec[None, :, :]
    x_ref[...] = jnp.tanh(comb).reshape(tt * u, in_f).astype(jnp.bfloat16)
    bias = bf_ref[0] - c0

    ch = 512
    tch = ch // u

    def _chunk(i, carry):
        xc = x_ref[pl.ds(i * ch, ch), :]
        logits = jax.lax.dot_general(
            xc, wft_ref[...], (((1,), (0,)), ((), ())),
            preferred_element_type=jnp.float32) + bias
        lse = jnp.log(jnp.sum(jnp.exp(logits), axis=-1, keepdims=True))
        out_ref[0, pl.ds(i * tch, tch)] = (logits - lse).reshape(tch, u, v)
        return carry

    jax.lax.fori_loop(0, (tt * u) // ch, _chunk, 0, unroll=False)


def kernel(encoder_output, decoder_output, W_enc, b_enc, W_dec, b_dec,
           W_fc, b_fc):
    B, T, E = encoder_output.shape
    _, U, D = decoder_output.shape
    in_f = W_enc.shape[0]
    V = W_fc.shape[0]
    tt = _TT

    grid = (B, T // tt)
    out = pl.pallas_call(
        _body,
        grid=grid,
        in_specs=[
            pl.BlockSpec((1, tt, E), lambda b, t: (b, t, 0)),
            pl.BlockSpec((1, U, D), lambda b, t: (b, 0, 0)),
            pl.BlockSpec((in_f, E), lambda b, t: (0, 0)),
            pl.BlockSpec((1, in_f), lambda b, t: (0, 0)),
            pl.BlockSpec((in_f, D), lambda b, t: (0, 0)),
            pl.BlockSpec((1, in_f), lambda b, t: (0, 0)),
            pl.BlockSpec((V, in_f), lambda b, t: (0, 0)),
            pl.BlockSpec((1, V), lambda b, t: (0, 0)),
        ],
        out_specs=pl.BlockSpec((1, tt, U, V), lambda b, t: (b, t, 0, 0)),
        out_shape=jax.ShapeDtypeStruct((B, T, U, V), jnp.float32),
        scratch_shapes=[pltpu.SMEM((1, 1), jnp.float32),
                        pltpu.VMEM((in_f, V), jnp.bfloat16),
                        pltpu.VMEM((tt * U, in_f), jnp.bfloat16)],
    )(
        encoder_output,
        decoder_output,
        W_enc,
        b_enc.reshape(1, in_f),
        W_dec,
        b_dec.reshape(1, in_f),
        W_fc,
        b_fc.reshape(1, V),
    )
    return out


# speedup vs baseline: 1.2642x; 1.2642x over previous
import jax
import jax.numpy as jnp
from jax.experimental import pallas as pl
from jax.experimental.pallas import tpu as pltpu

_TT = 128


def _body(enc_ref, dec_ref, we_ref, be_ref, wd_ref, bd_ref, wf_ref, bf_ref,
          out_ref, c0_ref, wft_ref):
    tt, e = enc_ref.shape[1], enc_ref.shape[2]
    u = dec_ref.shape[1]
    in_f = we_ref.shape[0]
    v = wf_ref.shape[0]

    first = (pl.program_id(0) == 0) & (pl.program_id(1) == 0)

    @pl.when(first)
    def _init():
        bound = jnp.sum(jnp.abs(wf_ref[...]), axis=1) + jnp.abs(bf_ref[0])
        c0_ref[0, 0] = jnp.max(bound)
        wft_ref[...] = jnp.transpose(wf_ref[...], (1, 0)).astype(jnp.bfloat16)

    c0 = c0_ref[0, 0]

    enc_x = enc_ref[0]
    dec_x = dec_ref[0]
    enc = jax.lax.dot_general(
        enc_x, we_ref[...], (((1,), (1,)), ((), ())),
        preferred_element_type=jnp.float32) + be_ref[0]
    dec = jax.lax.dot_general(
        dec_x, wd_ref[...], (((1,), (1,)), ((), ())),
        preferred_element_type=jnp.float32) + bd_ref[0]

    comb = enc[:, None, :] + dec[None, :, :]
    x = jnp.tanh(comb).reshape(tt * u, in_f).astype(jnp.bfloat16)
    logits = jax.lax.dot_general(
        x, wft_ref[...], (((1,), (0,)), ((), ())),
        preferred_element_type=jnp.float32) + (bf_ref[0] - c0)

    lse = jnp.log(jnp.sum(jnp.exp(logits), axis=-1, keepdims=True))
    out_ref[0] = (logits - lse).reshape(tt, u, v)


def kernel(encoder_output, decoder_output, W_enc, b_enc, W_dec, b_dec,
           W_fc, b_fc):
    B, T, E = encoder_output.shape
    _, U, D = decoder_output.shape
    in_f = W_enc.shape[0]
    V = W_fc.shape[0]
    tt = _TT

    grid = (B, T // tt)
    out = pl.pallas_call(
        _body,
        grid=grid,
        in_specs=[
            pl.BlockSpec((1, tt, E), lambda b, t: (b, t, 0)),
            pl.BlockSpec((1, U, D), lambda b, t: (b, 0, 0)),
            pl.BlockSpec((in_f, E), lambda b, t: (0, 0)),
            pl.BlockSpec((1, in_f), lambda b, t: (0, 0)),
            pl.BlockSpec((in_f, D), lambda b, t: (0, 0)),
            pl.BlockSpec((1, in_f), lambda b, t: (0, 0)),
            pl.BlockSpec((V, in_f), lambda b, t: (0, 0)),
            pl.BlockSpec((1, V), lambda b, t: (0, 0)),
        ],
        out_specs=pl.BlockSpec((1, tt, U, V), lambda b, t: (b, t, 0, 0)),
        out_shape=jax.ShapeDtypeStruct((B, T, U, V), jnp.float32),
        scratch_shapes=[pltpu.SMEM((1, 1), jnp.float32),
                        pltpu.VMEM((in_f, V), jnp.bfloat16)],
    )(
        encoder_output,
        decoder_output,
        W_enc,
        b_enc.reshape(1, in_f),
        W_dec,
        b_dec.reshape(1, in_f),
        W_fc,
        b_fc.reshape(1, V),
    )
    return out
